# baseline (speedup 1.0000x reference)
import functools

import jax
import jax.numpy as jnp
from jax import lax
from jax.experimental import pallas as pl
from jax.experimental.pallas import tpu as pltpu
from jax.experimental.pallas import tpu_sc as plsc

B = 4096
LG = 200
LS = 200
D = 128
KG = LG * D
NW = 32

NBG = 16
NG = B // NBG
DIM1 = NBG * LG
CHUNK = 128
CPG = DIM1 // CHUNK
GPW = NG // NW
NCH = GPW * CPG
NBUF = 4
GLAG = 2

SV = 1024
CH_S = 100
BPW = B // NW
NCH_S = BPW * 2


def _gene_phase(table, idx_v, out, bufs, gsems, wsems, g0):
    def g_copy(slot, j):
        return pltpu.make_async_copy(
            table.at[idx_v.at[j]], bufs.at[slot], gsems[slot])

    def w_copy(slot, j):
        off = pl.multiple_of((j % CPG) * CHUNK, 128)
        return pltpu.make_async_copy(
            bufs.at[slot],
            out.at[g0 + j // CPG, pl.ds(off, CHUNK)],
            wsems[slot])

    for s in range(GLAG):
        g_copy(s, s).start()

    def body(i, carry):
        for s in range(NBUF):
            j = i * NBUF + s
            g_copy(s, j).wait()
            w_copy(s, j).start()
            ns = (s + GLAG) % NBUF
            nxt = j + GLAG

            @pl.when(nxt < NCH)
            def _():
                @pl.when(nxt >= NBUF)
                def _():
                    w_copy(ns, nxt - NBUF).wait()
                g_copy(ns, nxt).start()
        return carry

    lax.fori_loop(0, NCH // NBUF, body, 0)
    for j in range(NCH - NBUF, NCH):
        w_copy(j % NBUF, j).wait()


def _smiles_phase(p_tab, idx_v, sp_out, bufs, gsems, accbuf, b0):
    def g_copy(slot, j):
        return pltpu.make_async_copy(
            p_tab.at[idx_v.at[j]], bufs.at[slot], gsems[slot])

    def _sum_chunk(slot, acc):
        def rbody(r, a):
            return (a[0] + bufs[slot, r, pl.ds(0, 16)],
                    a[1] + bufs[slot, r, pl.ds(16, 16)],
                    a[2] + bufs[slot, r, pl.ds(32, 16)],
                    a[3] + bufs[slot, r, pl.ds(48, 16)])
        return lax.fori_loop(0, CH_S, rbody, acc)

    for s in range(NBUF):
        g_copy(s, s).start()

    def body(i, carry):
        for h in range(2):
            zero = jnp.zeros((16,), jnp.float32)
            acc = (zero, zero, zero, zero)
            for s2 in range(2):
                s = 2 * h + s2
                j = 4 * i + s
                g_copy(s, j).wait()

                @pl.when(j + NBUF < NCH_S)
                def _():
                    g_copy(s, j + NBUF).start()

                acc = _sum_chunk(s, acc)
            for q in range(4):
                accbuf[2 * i + h, pl.ds(16 * q, 16)] = acc[q]
        return carry

    lax.fori_loop(0, NCH_S // NBUF, body, 0)
    pltpu.sync_copy(accbuf, sp_out.at[pl.ds(b0, BPW)])


@functools.partial(
    pl.kernel,
    out_type=(
        jax.ShapeDtypeStruct((NG, DIM1, D), jnp.float32),
        jax.ShapeDtypeStruct((B, 64), jnp.float32),
    ),
    mesh=plsc.VectorSubcoreMesh(core_axis_name="c", subcore_axis_name="s"),
    scratch_types=[
        pltpu.VMEM((NCH_S, CHUNK), jnp.int32),
        pltpu.VMEM((NBUF, CHUNK, D), jnp.float32),
        pltpu.VMEM((BPW, 64), jnp.float32),
        pltpu.SemaphoreType.DMA,
        pltpu.SemaphoreType.DMA,
        pltpu.SemaphoreType.DMA,
        pltpu.SemaphoreType.DMA,
        pltpu.SemaphoreType.DMA,
        pltpu.SemaphoreType.DMA,
        pltpu.SemaphoreType.DMA,
        pltpu.SemaphoreType.DMA,
    ],
)
def _sc_gather(gene_table, gene_idx, p_tab, smiles_idx,
               ge_out, sp_out, idx_v, bufs, accbuf,
               g0, g1, g2, g3, w0, w1, w2, w3):
    wid = lax.axis_index("c") * 16 + lax.axis_index("s")
    gsems = [g0, g1, g2, g3]
    wsems = [w0, w1, w2, w3]
    pltpu.sync_copy(gene_idx.at[pl.ds(wid * NCH, NCH)], idx_v.at[pl.ds(0, NCH)])
    _gene_phase(gene_table, idx_v, ge_out, bufs, gsems, wsems, wid * GPW)
    pltpu.sync_copy(smiles_idx.at[pl.ds(wid * NCH_S, NCH_S)], idx_v)
    _smiles_phase(p_tab, idx_v, sp_out, bufs, gsems, accbuf, wid * BPW)


TPP = 25


def _p_body(st, w1s, out):
    x = st[...]
    for t in range(TPP):
        out[pl.ds(t * SV, SV), :] = jnp.dot(
            x, w1s[pl.ds(t * D, D), :], preferred_element_type=jnp.float32)


def _p_compute(st_pad, w1s):
    return pl.pallas_call(
        _p_body,
        grid=(LS // TPP,),
        in_specs=[
            pl.BlockSpec((SV, D), lambda t: (0, 0)),
            pl.BlockSpec((TPP * D, D), lambda t: (t, 0)),
        ],
        out_specs=pl.BlockSpec((TPP * SV, D), lambda t: (t, 0)),
        out_shape=jax.ShapeDtypeStruct((LS * SV, D), jnp.float32),
    )(st_pad, w1s)


def _mlp_body(ge, sp, w1g, b1, w2, b2, w3, b3, out):
    gb = ge.shape[0]
    xg = ge[...].reshape(gb * NBG, KG)
    h = jnp.dot(xg, w1g[...], preferred_element_type=jnp.float32)
    h = jnp.maximum(h + sp[...] + b1[...], 0.0)
    h = jnp.maximum(
        jnp.dot(h, w2[...], preferred_element_type=jnp.float32) + b2[...],
        0.0)
    x = jnp.dot(h, w3[...], preferred_element_type=jnp.float32) + b3[...]
    out[...] = 1.0 / (1.0 + jnp.exp(-x))


def _mlp(ge, sp, w1g, b1, w2, b2, w3, b3):
    GB = 8
    grid = (NG // GB,)
    return pl.pallas_call(
        _mlp_body,
        grid=grid,
        in_specs=[
            pl.BlockSpec((GB, DIM1, D), lambda b: (b, 0, 0)),
            pl.BlockSpec((GB * NBG, 64), lambda b: (b, 0)),
            pl.BlockSpec((KG, 64), lambda b: (0, 0)),
            pl.BlockSpec((1, 64), lambda b: (0, 0)),
            pl.BlockSpec((64, 32), lambda b: (0, 0)),
            pl.BlockSpec((1, 32), lambda b: (0, 0)),
            pl.BlockSpec((32, 1), lambda b: (0, 0)),
            pl.BlockSpec((1, 1), lambda b: (0, 0)),
        ],
        out_specs=pl.BlockSpec((GB * NBG, 1), lambda b: (b, 0)),
        out_shape=jax.ShapeDtypeStruct((B, 1), jnp.float32),
    )(ge, sp, w1g, b1, w2, b2, w3, b3)


def kernel(gene_input, smiles_input, gene_table, smiles_table,
           W1, b1, W2, b2, W3, b3):
    gidx = gene_input.reshape(B * LG // CHUNK, CHUNK)
    st_pad = jnp.pad(smiles_table, ((0, SV - smiles_table.shape[0]), (0, 0)))
    w1s_pad = jnp.pad(W1[KG:], ((0, 0), (0, D - 64)))
    p_tab = _p_compute(st_pad, w1s_pad)
    sidx = (smiles_input
            + jnp.arange(LS, dtype=jnp.int32)[None, :] * SV)
    sidx = jnp.pad(sidx.reshape(B * 2, CH_S), ((0, 0), (0, CHUNK - CH_S)))
    ge, sp = _sc_gather(gene_table, gidx, p_tab, sidx)
    return _mlp(ge, sp, W1[:KG],
                b1.reshape(1, 64), W2, b2.reshape(1, 32),
                W3, b3.reshape(1, 1))


# speedup vs baseline: 1.0006x; 1.0006x over previous
import functools

import jax
import jax.numpy as jnp
from jax import lax
from jax.experimental import pallas as pl
from jax.experimental.pallas import tpu as pltpu
from jax.experimental.pallas import tpu_sc as plsc

B = 4096
LG = 200
LS = 200
D = 128
KG = LG * D
NW = 32

NBG = 16
NG = B // NBG
DIM1 = NBG * LG
CHUNK = 128
CPG = DIM1 // CHUNK
GPW = NG // NW
NCH = GPW * CPG
NBUF = 2

SV = 1024
CH_S = 100
BPW = B // NW
NCH_S = BPW * 2


def _gene_phase(table, idx_v, out, bufs, gsems, wsems, g0):
    def g_copy(slot, j):
        return pltpu.make_async_copy(
            table.at[idx_v.at[j]], bufs.at[slot], gsems[slot])

    def w_copy(slot, j):
        off = pl.multiple_of((j % CPG) * CHUNK, 128)
        return pltpu.make_async_copy(
            bufs.at[slot],
            out.at[g0 + j // CPG, pl.ds(off, CHUNK)],
            wsems[slot])

    for s in range(NBUF):
        g_copy(s, s).start()

    def body(i, carry):
        for s in range(NBUF):
            j = i * NBUF + s
            g_copy(s, j).wait()
            w_copy(s, j).start()
            w_copy(s, j).wait()

            @pl.when(j + NBUF < NCH)
            def _():
                g_copy(s, j + NBUF).start()
        return carry

    lax.fori_loop(0, NCH // NBUF, body, 0)


def _smiles_phase(p_tab, idx_v, sp_out, bufs, gsems, accbuf, b0):
    def g_copy(slot, j):
        return pltpu.make_async_copy(
            p_tab.at[idx_v.at[j]], bufs.at[slot], gsems[slot])

    def _sum_chunk(slot, acc):
        def rbody(r, a):
            return (a[0] + bufs[slot, r, pl.ds(0, 16)],
                    a[1] + bufs[slot, r, pl.ds(16, 16)],
                    a[2] + bufs[slot, r, pl.ds(32, 16)],
                    a[3] + bufs[slot, r, pl.ds(48, 16)])
        return lax.fori_loop(0, CH_S, rbody, acc)

    for s in range(NBUF):
        g_copy(s, s).start()

    def body(i, carry):
        zero = jnp.zeros((16,), jnp.float32)
        acc = (zero, zero, zero, zero)
        for s in range(NBUF):
            j = 2 * i + s
            g_copy(s, j).wait()

            @pl.when(j + NBUF < NCH_S)
            def _():
                g_copy(s, j + NBUF).start()

            acc = _sum_chunk(s, acc)
        for q in range(4):
            accbuf[i, pl.ds(16 * q, 16)] = acc[q]
        return carry

    lax.fori_loop(0, NCH_S // 2, body, 0)
    pltpu.sync_copy(accbuf, sp_out.at[pl.ds(b0, BPW)])


@functools.partial(
    pl.kernel,
    out_type=(
        jax.ShapeDtypeStruct((NG, DIM1, D), jnp.float32),
        jax.ShapeDtypeStruct((B, 64), jnp.float32),
    ),
    mesh=plsc.VectorSubcoreMesh(core_axis_name="c", subcore_axis_name="s"),
    scratch_types=[
        pltpu.VMEM((NCH_S, CHUNK), jnp.int32),
        pltpu.VMEM((NBUF, CHUNK, D), jnp.float32),
        pltpu.VMEM((BPW, 64), jnp.float32),
        pltpu.SemaphoreType.DMA,
        pltpu.SemaphoreType.DMA,
        pltpu.SemaphoreType.DMA,
        pltpu.SemaphoreType.DMA,
    ],
)
def _sc_gather(gene_table, gene_idx, p_tab, smiles_idx,
               ge_out, sp_out, idx_v, bufs, accbuf,
               g0, g1, w0, w1):
    wid = lax.axis_index("c") * 16 + lax.axis_index("s")
    gsems = [g0, g1]
    wsems = [w0, w1]
    pltpu.sync_copy(gene_idx.at[pl.ds(wid * NCH, NCH)], idx_v.at[pl.ds(0, NCH)])
    _gene_phase(gene_table, idx_v, ge_out, bufs, gsems, wsems, wid * GPW)
    pltpu.sync_copy(smiles_idx.at[pl.ds(wid * NCH_S, NCH_S)], idx_v)
    _smiles_phase(p_tab, idx_v, sp_out, bufs, gsems, accbuf, wid * BPW)


TPP = 25


def _p_body(st, w1s, out):
    x = st[...]
    for t in range(TPP):
        out[pl.ds(t * SV, SV), :] = jnp.dot(
            x, w1s[pl.ds(t * D, D), :], preferred_element_type=jnp.float32)


def _p_compute(st_pad, w1s):
    return pl.pallas_call(
        _p_body,
        grid=(LS // TPP,),
        in_specs=[
            pl.BlockSpec((SV, D), lambda t: (0, 0)),
            pl.BlockSpec((TPP * D, D), lambda t: (t, 0)),
        ],
        out_specs=pl.BlockSpec((TPP * SV, D), lambda t: (t, 0)),
        out_shape=jax.ShapeDtypeStruct((LS * SV, D), jnp.float32),
    )(st_pad, w1s)


def _mlp_body(ge, sp, w1g, b1, w2, b2, w3, b3, out):
    gb = ge.shape[0]
    xg = ge[...].reshape(gb * NBG, KG)
    h = jnp.dot(xg, w1g[...], preferred_element_type=jnp.float32)
    h = jnp.maximum(h + sp[...] + b1[...], 0.0)
    h = jnp.maximum(
        jnp.dot(h, w2[...], preferred_element_type=jnp.float32) + b2[...],
        0.0)
    x = jnp.dot(h, w3[...], preferred_element_type=jnp.float32) + b3[...]
    out[...] = 1.0 / (1.0 + jnp.exp(-x))


def _mlp(ge, sp, w1g, b1, w2, b2, w3, b3):
    GB = 8
    grid = (NG // GB,)
    return pl.pallas_call(
        _mlp_body,
        grid=grid,
        in_specs=[
            pl.BlockSpec((GB, DIM1, D), lambda b: (b, 0, 0)),
            pl.BlockSpec((GB * NBG, 64), lambda b: (b, 0)),
            pl.BlockSpec((KG, 64), lambda b: (0, 0)),
            pl.BlockSpec((1, 64), lambda b: (0, 0)),
            pl.BlockSpec((64, 32), lambda b: (0, 0)),
            pl.BlockSpec((1, 32), lambda b: (0, 0)),
            pl.BlockSpec((32, 1), lambda b: (0, 0)),
            pl.BlockSpec((1, 1), lambda b: (0, 0)),
        ],
        out_specs=pl.BlockSpec((GB * NBG, 1), lambda b: (b, 0)),
        out_shape=jax.ShapeDtypeStruct((B, 1), jnp.float32),
    )(ge, sp, w1g, b1, w2, b2, w3, b3)


def kernel(gene_input, smiles_input, gene_table, smiles_table,
           W1, b1, W2, b2, W3, b3):
    gidx = gene_input.reshape(B * LG // CHUNK, CHUNK)
    st_pad = jnp.pad(smiles_table, ((0, SV - smiles_table.shape[0]), (0, 0)))
    w1s_pad = jnp.pad(W1[KG:], ((0, 0), (0, D - 64)))
    p_tab = _p_compute(st_pad, w1s_pad)
    sidx = (smiles_input
            + jnp.arange(LS, dtype=jnp.int32)[None, :] * SV)
    sidx = jnp.pad(sidx.reshape(B * 2, CH_S), ((0, 0), (0, CHUNK - CH_S)))
    ge, sp = _sc_gather(gene_table, gidx, p_tab, sidx)
    return _mlp(ge, sp, W1[:KG],
                b1.reshape(1, 64), W2, b2.reshape(1, 32),
                W3, b3.reshape(1, 1))


# speedup vs baseline: 12.9529x; 12.9447x over previous
import functools

import jax
import jax.numpy as jnp
from jax import lax
from jax.experimental import pallas as pl
from jax.experimental.pallas import tpu as pltpu
from jax.experimental.pallas import tpu_sc as plsc

B = 4096
LG = 200
LS = 200
D = 128
KG = LG * D
NW = 32

NBG = 16
NG = B // NBG
DIM1 = NBG * LG
CHUNK = 128
CPG = DIM1 // CHUNK
GPW = NG // NW
NCH = GPW * CPG
NBUF = 2

SV = 1024
CH_S = 100
BPW = B // NW
NCH_S = BPW * 2


def _gene_phase(table, idx_v, out, bufs, gsems, wsems, g0):
    def g_copy(slot, j):
        return pltpu.make_async_copy(
            table.at[idx_v.at[j]], bufs.at[slot], gsems[slot])

    def w_copy(slot, j):
        off = pl.multiple_of((j % CPG) * CHUNK, 128)
        return pltpu.make_async_copy(
            bufs.at[slot],
            out.at[g0 + j // CPG, pl.ds(off, CHUNK)],
            wsems[slot])

    for s in range(NBUF):
        g_copy(s, s).start()

    def body(i, carry):
        for s in range(NBUF):
            j = i * NBUF + s
            g_copy(s, j).wait()
            w_copy(s, j).start()
            w_copy(s, j).wait()

            @pl.when(j + NBUF < NCH)
            def _():
                g_copy(s, j + NBUF).start()
        return carry

    lax.fori_loop(0, NCH // NBUF, body, 0)


def _smiles_phase(p_tab, idx_v, sp_out, bufs, gsems, accbuf, b0):
    def g_copy(slot, j):
        return pltpu.make_async_copy(
            p_tab.at[idx_v.at[j]], bufs.at[slot, pl.ds(0, CH_S)],
            gsems[slot])

    def _sum_chunk(slot, acc):
        def rbody(r, a):
            return (a[0] + bufs[slot, r, pl.ds(0, 16)],
                    a[1] + bufs[slot, r, pl.ds(16, 16)],
                    a[2] + bufs[slot, r, pl.ds(32, 16)],
                    a[3] + bufs[slot, r, pl.ds(48, 16)])
        return lax.fori_loop(0, CH_S, rbody, acc)

    for s in range(NBUF):
        g_copy(s, s).start()

    def body(i, carry):
        zero = jnp.zeros((16,), jnp.float32)
        acc = (zero, zero, zero, zero)
        for s in range(NBUF):
            j = 2 * i + s
            g_copy(s, j).wait()

            @pl.when(j + NBUF < NCH_S)
            def _():
                g_copy(s, j + NBUF).start()

            acc = _sum_chunk(s, acc)
        for q in range(4):
            accbuf[i, pl.ds(16 * q, 16)] = acc[q]
        return carry

    lax.fori_loop(0, NCH_S // 2, body, 0)
    pltpu.sync_copy(accbuf, sp_out.at[pl.ds(b0, BPW)])


@functools.partial(
    pl.kernel,
    out_type=(
        jax.ShapeDtypeStruct((NG, DIM1, D), jnp.float32),
        jax.ShapeDtypeStruct((B, 64), jnp.float32),
    ),
    mesh=plsc.VectorSubcoreMesh(core_axis_name="c", subcore_axis_name="s"),
    scratch_types=[
        pltpu.VMEM((NCH, CHUNK), jnp.int32),
        pltpu.VMEM((NCH_S, CH_S), jnp.int32),
        pltpu.VMEM((NBUF, CHUNK, D), jnp.float32),
        pltpu.VMEM((BPW, 64), jnp.float32),
        pltpu.SemaphoreType.DMA,
        pltpu.SemaphoreType.DMA,
        pltpu.SemaphoreType.DMA,
        pltpu.SemaphoreType.DMA,
    ],
)
def _sc_gather(gene_table, gene_idx, p_tab, smiles_idx,
               ge_out, sp_out, idx_v, idx_s, bufs, accbuf,
               g0, g1, w0, w1):
    wid = lax.axis_index("c") * 16 + lax.axis_index("s")
    gsems = [g0, g1]
    wsems = [w0, w1]
    pltpu.sync_copy(gene_idx.at[pl.ds(wid * NCH, NCH)], idx_v)
    _gene_phase(gene_table, idx_v, ge_out, bufs, gsems, wsems, wid * GPW)
    pltpu.sync_copy(smiles_idx.at[pl.ds(wid * NCH_S, NCH_S)], idx_s)
    _smiles_phase(p_tab, idx_s, sp_out, bufs, gsems, accbuf, wid * BPW)


TPP = 25


def _p_body(st, w1s, out):
    x = st[...]
    for t in range(TPP):
        out[pl.ds(t * SV, SV), :] = jnp.dot(
            x, w1s[pl.ds(t * D, D), :], preferred_element_type=jnp.float32)


def _p_compute(st_pad, w1s):
    return pl.pallas_call(
        _p_body,
        grid=(LS // TPP,),
        in_specs=[
            pl.BlockSpec((SV, D), lambda t: (0, 0)),
            pl.BlockSpec((TPP * D, D), lambda t: (t, 0)),
        ],
        out_specs=pl.BlockSpec((TPP * SV, D), lambda t: (t, 0)),
        out_shape=jax.ShapeDtypeStruct((LS * SV, D), jnp.float32),
    )(st_pad, w1s)


def _mlp_body(ge, sp, w1g, b1, w2, b2, w3, b3, out):
    gb = ge.shape[0]
    xg = ge[...].reshape(gb * NBG, KG)
    h = jnp.dot(xg, w1g[...], preferred_element_type=jnp.float32)
    h = jnp.maximum(h + sp[...] + b1[...], 0.0)
    h = jnp.maximum(
        jnp.dot(h, w2[...], preferred_element_type=jnp.float32) + b2[...],
        0.0)
    x = jnp.dot(h, w3[...], preferred_element_type=jnp.float32) + b3[...]
    out[...] = 1.0 / (1.0 + jnp.exp(-x))


def _mlp(ge, sp, w1g, b1, w2, b2, w3, b3):
    GB = 8
    grid = (NG // GB,)
    return pl.pallas_call(
        _mlp_body,
        grid=grid,
        in_specs=[
            pl.BlockSpec((GB, DIM1, D), lambda b: (b, 0, 0)),
            pl.BlockSpec((GB * NBG, 64), lambda b: (b, 0)),
            pl.BlockSpec((KG, 64), lambda b: (0, 0)),
            pl.BlockSpec((1, 64), lambda b: (0, 0)),
            pl.BlockSpec((64, 32), lambda b: (0, 0)),
            pl.BlockSpec((1, 32), lambda b: (0, 0)),
            pl.BlockSpec((32, 1), lambda b: (0, 0)),
            pl.BlockSpec((1, 1), lambda b: (0, 0)),
        ],
        out_specs=pl.BlockSpec((GB * NBG, 1), lambda b: (b, 0)),
        out_shape=jax.ShapeDtypeStruct((B, 1), jnp.float32),
    )(ge, sp, w1g, b1, w2, b2, w3, b3)


def kernel(gene_input, smiles_input, gene_table, smiles_table,
           W1, b1, W2, b2, W3, b3):
    gidx = gene_input.reshape(B * LG // CHUNK, CHUNK)
    st_pad = jnp.pad(smiles_table, ((0, SV - smiles_table.shape[0]), (0, 0)))
    w1s_pad = jnp.pad(W1[KG:], ((0, 0), (0, D - 64)))
    p_tab = _p_compute(st_pad, w1s_pad)
    sidx = (smiles_input
            + jnp.arange(LS, dtype=jnp.int32)[None, :] * SV)
    sidx = sidx.reshape(B * 2, CH_S)
    ge, sp = _sc_gather(gene_table, gidx, p_tab, sidx)
    return _mlp(ge, sp, W1[:KG],
                b1.reshape(1, 64), W2, b2.reshape(1, 32),
                W3, b3.reshape(1, 1))


# speedup vs baseline: 13.3744x; 1.0325x over previous
import functools

import jax
import jax.numpy as jnp
from jax import lax
from jax.experimental import pallas as pl
from jax.experimental.pallas import tpu as pltpu
from jax.experimental.pallas import tpu_sc as plsc

B = 4096
LG = 200
LS = 200
D = 128
KG = LG * D
NW = 32

NBG = 16
NG = B // NBG
DIM1 = NBG * LG
CHUNK = 128
CPG = DIM1 // CHUNK
GPW = NG // NW
NCH = GPW * CPG
NBUF = 2

SV = 1024
CH_S = 100
BPW = B // NW
NCH_S = BPW * 2


def _gene_phase(table, idx_v, out, bufs, gsems, wsems, g0):
    def g_copy(slot, j):
        return pltpu.make_async_copy(
            table.at[idx_v.at[j]], bufs.at[slot], gsems[slot])

    def w_copy(slot, j):
        off = pl.multiple_of((j % CPG) * CHUNK, 128)
        return pltpu.make_async_copy(
            bufs.at[slot],
            out.at[g0 + j // CPG, pl.ds(off, CHUNK)],
            wsems[slot])

    for s in range(NBUF):
        g_copy(s, s).start()

    def body(i, carry):
        for s in range(NBUF):
            j = i * NBUF + s
            g_copy(s, j).wait()
            w_copy(s, j).start()
            w_copy(s, j).wait()

            @pl.when(j + NBUF < NCH)
            def _():
                g_copy(s, j + NBUF).start()
        return carry

    lax.fori_loop(0, NCH // NBUF, body, 0)


def _smiles_phase(p_tab, idx_v, sp_out, bufs, gsems, accbuf, b0):
    def g_copy(slot, j):
        return pltpu.make_async_copy(
            p_tab.at[idx_v.at[j]], bufs.at[slot], gsems[slot])

    def _sum_chunk(slot, acc):
        def rbody(r, a):
            return (a[0] + bufs[slot, r, pl.ds(0, 16)],
                    a[1] + bufs[slot, r, pl.ds(16, 16)],
                    a[2] + bufs[slot, r, pl.ds(32, 16)],
                    a[3] + bufs[slot, r, pl.ds(48, 16)])
        return lax.fori_loop(0, CH_S, rbody, acc)

    for s in range(NBUF):
        g_copy(s, s).start()

    def body(i, carry):
        zero = jnp.zeros((16,), jnp.float32)
        acc = (zero, zero, zero, zero)
        for s in range(NBUF):
            j = 2 * i + s
            g_copy(s, j).wait()

            @pl.when(j + NBUF < NCH_S)
            def _():
                g_copy(s, j + NBUF).start()

            acc = _sum_chunk(s, acc)
        for q in range(4):
            accbuf[i, pl.ds(16 * q, 16)] = acc[q]
        return carry

    lax.fori_loop(0, NCH_S // 2, body, 0)
    pltpu.sync_copy(accbuf, sp_out.at[pl.ds(b0, BPW)])


@functools.partial(
    pl.kernel,
    out_type=jax.ShapeDtypeStruct((NG, DIM1, D), jnp.float32),
    mesh=plsc.VectorSubcoreMesh(core_axis_name="c", subcore_axis_name="s"),
    scratch_types=[
        pltpu.VMEM((NCH, CHUNK), jnp.int32),
        pltpu.VMEM((NBUF, CHUNK, D), jnp.float32),
        pltpu.SemaphoreType.DMA,
        pltpu.SemaphoreType.DMA,
        pltpu.SemaphoreType.DMA,
        pltpu.SemaphoreType.DMA,
    ],
)
def _sc_gene(gene_table, gene_idx, ge_out, idx_v, bufs, g0, g1, w0, w1):
    wid = lax.axis_index("c") * 16 + lax.axis_index("s")
    pltpu.sync_copy(gene_idx.at[pl.ds(wid * NCH, NCH)], idx_v)
    _gene_phase(gene_table, idx_v, ge_out, bufs, [g0, g1], [w0, w1],
                wid * GPW)


@functools.partial(
    pl.kernel,
    out_type=jax.ShapeDtypeStruct((B, 64), jnp.float32),
    mesh=plsc.VectorSubcoreMesh(core_axis_name="c", subcore_axis_name="s"),
    scratch_types=[
        pltpu.VMEM((NCH_S, CH_S), jnp.int32),
        pltpu.VMEM((NBUF, CH_S, D), jnp.float32),
        pltpu.VMEM((BPW, 64), jnp.float32),
        pltpu.SemaphoreType.DMA,
        pltpu.SemaphoreType.DMA,
    ],
)
def _sc_smiles(p_tab, smiles_idx, sp_out, idx_s, bufs, accbuf, g0, g1):
    wid = lax.axis_index("c") * 16 + lax.axis_index("s")
    pltpu.sync_copy(smiles_idx.at[pl.ds(wid * NCH_S, NCH_S)], idx_s)
    _smiles_phase(p_tab, idx_s, sp_out, bufs, [g0, g1], accbuf, wid * BPW)


TPP = 25


def _p_body(st, w1s, out):
    x = st[...]
    z = jnp.zeros((SV, 64), jnp.float32)
    for t in range(TPP):
        r = jnp.dot(x, w1s[pl.ds(t * D, D), :],
                    preferred_element_type=jnp.float32)
        out[pl.ds(t * SV, SV), :] = jnp.concatenate([r, z], axis=1)


def _p_compute(st_pad, w1s):
    return pl.pallas_call(
        _p_body,
        grid=(LS // TPP,),
        in_specs=[
            pl.BlockSpec((SV, D), lambda t: (0, 0)),
            pl.BlockSpec((TPP * D, 64), lambda t: (t + LS // TPP, 0)),
        ],
        out_specs=pl.BlockSpec((TPP * SV, D), lambda t: (t, 0)),
        out_shape=jax.ShapeDtypeStruct((LS * SV, D), jnp.float32),
    )(st_pad, w1s)


def _mlp_body(ge, sp, w1g, b1, w2, b2, w3, b3, out):
    gb = ge.shape[0]
    xg = ge[...].reshape(gb * NBG, KG)
    h = jnp.dot(xg, w1g[...], preferred_element_type=jnp.float32)
    h = jnp.maximum(h + sp[...] + b1[...], 0.0)
    h = jnp.maximum(
        jnp.dot(h, w2[...], preferred_element_type=jnp.float32) + b2[...],
        0.0)
    x = jnp.dot(h, w3[...], preferred_element_type=jnp.float32) + b3[...]
    out[...] = 1.0 / (1.0 + jnp.exp(-x))


def _mlp(ge, sp, w1g, b1, w2, b2, w3, b3):
    GB = 8
    grid = (NG // GB,)
    return pl.pallas_call(
        _mlp_body,
        grid=grid,
        in_specs=[
            pl.BlockSpec((GB, DIM1, D), lambda b: (b, 0, 0)),
            pl.BlockSpec((GB * NBG, 64), lambda b: (b, 0)),
            pl.BlockSpec((KG, 64), lambda b: (0, 0)),
            pl.BlockSpec((1, 64), lambda b: (0, 0)),
            pl.BlockSpec((64, 32), lambda b: (0, 0)),
            pl.BlockSpec((1, 32), lambda b: (0, 0)),
            pl.BlockSpec((32, 1), lambda b: (0, 0)),
            pl.BlockSpec((1, 1), lambda b: (0, 0)),
        ],
        out_specs=pl.BlockSpec((GB * NBG, 1), lambda b: (b, 0)),
        out_shape=jax.ShapeDtypeStruct((B, 1), jnp.float32),
    )(ge, sp, w1g, b1, w2, b2, w3, b3)


def kernel(gene_input, smiles_input, gene_table, smiles_table,
           W1, b1, W2, b2, W3, b3):
    gidx = gene_input.reshape(B * LG // CHUNK, CHUNK)
    ge = _sc_gene(gene_table, gidx)
    st_pad = jnp.pad(smiles_table, ((0, SV - smiles_table.shape[0]), (0, 0)))
    p_tab = _p_compute(st_pad, W1)
    sidx = (smiles_input
            + jnp.arange(LS, dtype=jnp.int32)[None, :] * SV)
    sidx = sidx.reshape(B * 2, CH_S)
    sp = _sc_smiles(p_tab, sidx)
    return _mlp(ge, sp, W1,
                b1.reshape(1, 64), W2, b2.reshape(1, 32),
                W3, b3.reshape(1, 1))
